# baseline (speedup 1.0000x reference)
import functools

import jax
import jax.numpy as jnp
from jax import lax
from jax.experimental import pallas as pl
from jax.experimental.pallas import tpu as pltpu
from jax.experimental.pallas import tpu_sc as plsc

_B = 8
_C = 80
_N = 20000
_MAX_DET = 100
_PRE_TOPK = 512
_IOU_THR = 0.5
_SCORE_THR = 0.25

_NS = 8
_NL = _N // _NS
_LANES = 16


def _float_key(bits):
    return jnp.where(bits >= 0, bits, bits ^ jnp.int32(0x7FFFFFFF))


def _k1a_body(x_ref, maxsc_ref, cls_ref):
    xs = x_ref[0]
    ci8 = lax.broadcasted_iota(jnp.int32, (8, _N), 0)
    m8 = xs[4:12, :]
    c8 = ci8
    for g in range(1, 10):
        blk = xs[4 + 8 * g: 12 + 8 * g, :]
        upd = blk > m8
        c8 = jnp.where(upd, ci8 + 8 * g, c8)
        m8 = jnp.maximum(m8, blk)
    m4 = jnp.maximum(m8[:4], m8[4:])
    c4 = jnp.where(m8[:4] >= m8[4:], c8[:4], c8[4:])
    m2 = jnp.maximum(m4[:2], m4[2:])
    c2 = jnp.where(m4[:2] >= m4[2:], c4[:2], c4[2:])
    m1 = jnp.maximum(m2[:1], m2[1:])
    c1 = jnp.where(m2[:1] >= m2[1:], c2[:1], c2[1:])
    maxsc_ref[0] = m1
    cls_ref[0] = c1


def _k1a_call(x):
    return pl.pallas_call(
        _k1a_body,
        grid=(_B,),
        in_specs=[pl.BlockSpec((1, 4 + _C, _N), lambda b: (b, 0, 0))],
        out_specs=[
            pl.BlockSpec((1, 1, _N), lambda b: (b, 0, 0)),
            pl.BlockSpec((1, 1, _N), lambda b: (b, 0, 0)),
        ],
        out_shape=[
            jax.ShapeDtypeStruct((_B, 1, _N), jnp.float32),
            jax.ShapeDtypeStruct((_B, 1, _N), jnp.int32),
        ],
    )(x)


def _k1b_body(maxsc_ref, tau_ref, bound_ref):
    M = maxsc_ref[...]
    key = _float_key(lax.bitcast_convert_type(M, jnp.int32))
    kmin = jnp.min(key, axis=(1, 2), keepdims=True)
    kmax = jnp.max(key, axis=(1, 2), keepdims=True)

    def cnt_ge(v):
        return jnp.sum(jnp.where(key >= v, 1.0, 0.0), axis=(1, 2),
                       keepdims=True)

    topkf = float(_PRE_TOPK)

    def sbody(_, carry):
        lo, hi = carry
        mid = lo + (hi - lo) // 2
        p = cnt_ge(mid) >= topkf
        return jnp.where(p, mid, lo), jnp.where(p, hi, mid)

    lo, _hi = lax.fori_loop(0, 32, sbody, (kmin, kmax + 1))
    tau = lo
    n_tie = topkf - jnp.sum(jnp.where(key > tau, 1.0, 0.0), axis=(1, 2),
                            keepdims=True)

    flat = (lax.broadcasted_iota(jnp.int32, (_B, _NS, _NL), 1) * _NL
            + lax.broadcasted_iota(jnp.int32, (_B, _NS, _NL), 2))
    eqm = key == tau

    def tbody(_, carry):
        lo2, hi2 = carry
        mid = (lo2 + hi2) // 2
        cnt = jnp.sum(jnp.where(eqm & (flat < mid), 1.0, 0.0), axis=(1, 2),
                      keepdims=True)
        q = cnt >= n_tie
        return jnp.where(q, lo2, mid), jnp.where(q, mid, hi2)

    zero = jnp.zeros((_B, 1, 1), jnp.int32)
    _lo2, bound = lax.fori_loop(0, 15, tbody, (zero, zero + _N))

    tau_bits = _float_key(tau)
    tau_f = lax.bitcast_convert_type(tau_bits, jnp.float32)
    tau_ref[...] = jnp.broadcast_to(tau_f, (_B, 1, 16))
    bound_ref[...] = jnp.broadcast_to(bound, (_B, 1, 16))


def _k1b_call(maxsc):
    return pl.pallas_call(
        _k1b_body,
        out_shape=[
            jax.ShapeDtypeStruct((_B, 1, 16), jnp.float32),
            jax.ShapeDtypeStruct((_B, 1, 16), jnp.int32),
        ],
    )(maxsc)


def _k1_call(x):
    maxsc, cls8 = _k1a_call(x)
    maxsc = maxsc.reshape(_B, _NS, _NL)
    tau, bound = _k1b_call(maxsc)
    return maxsc, cls8, tau, bound


def _k2_body(maxsc_hbm, cls_hbm, x_hbm, tau_hbm, bnd_hbm,
             sc_out, cls_out, bx_out,
             sc_v, cls_v, cx_v, cy_v, w_v, h_v,
             tau_v, bnd_v, idx_v, osc_v, ocls_v, o0, o1, o2, o3):
    c = lax.axis_index("c")
    s = lax.axis_index("s")
    wid = s * 2 + c

    @pl.when(wid < _B)
    def _():
        b = wid
        pltpu.sync_copy(maxsc_hbm.at[b], sc_v)
        pltpu.sync_copy(cls_hbm.at[b], cls_v)
        pltpu.sync_copy(x_hbm.at[b, 0], cx_v)
        pltpu.sync_copy(x_hbm.at[b, 1], cy_v)
        pltpu.sync_copy(x_hbm.at[b, 2], w_v)
        pltpu.sync_copy(x_hbm.at[b, 3], h_v)
        pltpu.sync_copy(tau_hbm.at[b], tau_v)
        pltpu.sync_copy(bnd_hbm.at[b], bnd_v)
        tau = tau_v[...]
        bndf = bnd_v[...].astype(jnp.float32)
        lane = lax.iota(jnp.int32, _LANES)

        def body(i, cur):
            v = sc_v[pl.ds(i * _LANES, _LANES)]
            cl = cls_v[pl.ds(i * _LANES, _LANES)]
            idx = lane + i * _LANES
            idxf = idx.astype(jnp.float32)
            sel = (v > tau) | ((v == tau) & (idxf < bndf))
            csum = plsc.cumsum(sel.astype(jnp.int32))
            pos = csum + (cur - 1)
            plsc.store_scatter(idx_v, [pos], idx, mask=sel)
            plsc.store_scatter(osc_v, [pos], v, mask=sel)
            plsc.store_scatter(ocls_v, [pos], cl, mask=sel)
            return cur + jnp.max(csum)

        lax.fori_loop(0, _N // _LANES, body, jnp.int32(0), unroll=4)

        def gbody(i, _):
            sl = pl.ds(i * _LANES, _LANES)
            ii = idx_v[sl]
            cx = plsc.load_gather(cx_v, [ii])
            cy = plsc.load_gather(cy_v, [ii])
            w = plsc.load_gather(w_v, [ii])
            h = plsc.load_gather(h_v, [ii])
            o0[sl] = cx - w * 0.5
            o1[sl] = cy - h * 0.5
            o2[sl] = cx + w * 0.5
            o3[sl] = cy + h * 0.5
            return 0

        lax.fori_loop(0, _PRE_TOPK // _LANES, gbody, 0, unroll=4)

        pltpu.sync_copy(osc_v, sc_out.at[b])
        pltpu.sync_copy(ocls_v, cls_out.at[b])
        pltpu.sync_copy(o0, bx_out.at[b, 0])
        pltpu.sync_copy(o1, bx_out.at[b, 1])
        pltpu.sync_copy(o2, bx_out.at[b, 2])
        pltpu.sync_copy(o3, bx_out.at[b, 3])


def _k2_call(maxsc, cls8, x, tau, bound):
    mesh = plsc.VectorSubcoreMesh(core_axis_name="c", subcore_axis_name="s")
    f = functools.partial(
        pl.kernel,
        out_type=[
            jax.ShapeDtypeStruct((_B, _PRE_TOPK), jnp.float32),
            jax.ShapeDtypeStruct((_B, _PRE_TOPK), jnp.int32),
            jax.ShapeDtypeStruct((_B, 4, _PRE_TOPK), jnp.float32),
        ],
        mesh=mesh,
        compiler_params=pltpu.CompilerParams(needs_layout_passes=False),
        scratch_types=[
            pltpu.VMEM((_N,), jnp.float32),
            pltpu.VMEM((_N,), jnp.int32),
            pltpu.VMEM((_N,), jnp.float32),
            pltpu.VMEM((_N,), jnp.float32),
            pltpu.VMEM((_N,), jnp.float32),
            pltpu.VMEM((_N,), jnp.float32),
            pltpu.VMEM((16,), jnp.float32),
            pltpu.VMEM((16,), jnp.int32),
            pltpu.VMEM((_PRE_TOPK,), jnp.int32),
            pltpu.VMEM((_PRE_TOPK,), jnp.float32),
            pltpu.VMEM((_PRE_TOPK,), jnp.int32),
            pltpu.VMEM((_PRE_TOPK,), jnp.float32),
            pltpu.VMEM((_PRE_TOPK,), jnp.float32),
            pltpu.VMEM((_PRE_TOPK,), jnp.float32),
            pltpu.VMEM((_PRE_TOPK,), jnp.float32),
        ],
    )(_k2_body)
    return f(maxsc, cls8, x, tau, bound)


def _k3_body(sc_ref, cls_ref, bx_ref, nd_ref, db_ref, ds_ref, dc_ref):
    sc = sc_ref[...]
    clsf = cls_ref[...].astype(jnp.float32)
    x1 = bx_ref[:, 0, :]
    y1 = bx_ref[:, 1, :]
    x2 = bx_ref[:, 2, :]
    y2 = bx_ref[:, 3, :]
    area = jnp.clip(x2 - x1, 0.0) * jnp.clip(y2 - y1, 0.0)
    P = jnp.concatenate([x1, y1, x2, y2, clsf], axis=0)

    sc_w0 = jnp.where(sc > _SCORE_THR, sc, -1.0)
    iota = lax.broadcasted_iota(jnp.int32, (_B, _PRE_TOPK), 1)
    iota40 = lax.broadcasted_iota(jnp.int32, (5 * _B, _PRE_TOPK), 1)
    iota_o = lax.broadcasted_iota(jnp.int32, (_B, 128), 1)
    zf = jnp.zeros((_B, 128), jnp.float32)

    def body(i, carry):
        sc_w, cnt, a1o, a2o, a3o, a4o, aso, aco = carry
        m = jnp.max(sc_w, axis=1, keepdims=True)
        eq = sc_w == m
        j = jnp.min(jnp.where(eq, iota, _PRE_TOPK), axis=1, keepdims=True)
        j5 = jnp.concatenate([j, j, j, j, j], axis=0)
        oh40 = iota40 == j5
        pays = jnp.sum(jnp.where(oh40, P, 0.0), axis=1, keepdims=True)
        bx1 = pays[0:8]
        by1 = pays[8:16]
        bx2 = pays[16:24]
        by2 = pays[24:32]
        bcf = pays[32:40]
        keep = m > _SCORE_THR
        ohw = (iota_o == i) & keep
        a1o = jnp.where(ohw, bx1, a1o)
        a2o = jnp.where(ohw, by1, a2o)
        a3o = jnp.where(ohw, bx2, a3o)
        a4o = jnp.where(ohw, by2, a4o)
        aso = jnp.where(ohw, m, aso)
        aco = jnp.where(ohw, bcf, aco)
        cnt = cnt + keep.astype(jnp.int32)
        ix1 = jnp.maximum(bx1, x1)
        iy1 = jnp.maximum(by1, y1)
        ix2 = jnp.minimum(bx2, x2)
        iy2 = jnp.minimum(by2, y2)
        inter = jnp.clip(ix2 - ix1, 0.0) * jnp.clip(iy2 - iy1, 0.0)
        a1 = jnp.clip(bx2 - bx1, 0.0) * jnp.clip(by2 - by1, 0.0)
        iou = inter / (a1 + area - inter + 1e-9)
        supp = (iou > _IOU_THR) & (clsf == bcf)
        sc_w = jnp.where(supp | oh40[0:8], -1.0, sc_w)
        return sc_w, cnt, a1o, a2o, a3o, a4o, aso, aco

    init = (sc_w0, jnp.zeros((_B, 1), jnp.int32), zf, zf, zf, zf, zf,
            zf - 1.0)
    _, cnt, a1o, a2o, a3o, a4o, aso, aco = lax.fori_loop(
        0, _MAX_DET, body, init)
    nd_ref[...] = cnt
    db_ref[...] = jnp.concatenate(
        [a1o[:, None, :], a2o[:, None, :], a3o[:, None, :], a4o[:, None, :]],
        axis=1)
    ds_ref[...] = aso
    dc_ref[...] = aco.astype(jnp.int32)


def _k3_call(sc512, cls512, bx):
    return pl.pallas_call(
        _k3_body,
        out_shape=[
            jax.ShapeDtypeStruct((_B, 1), jnp.int32),
            jax.ShapeDtypeStruct((_B, 4, 128), jnp.float32),
            jax.ShapeDtypeStruct((_B, 128), jnp.float32),
            jax.ShapeDtypeStruct((_B, 128), jnp.int32),
        ],
    )(sc512, cls512, bx)


def kernel(x):
    maxsc, cls8, tau, bound = _k1_call(x)
    sc512, cls512, bx = _k2_call(
        maxsc.reshape(_B, _N), cls8.reshape(_B, _N), x,
        tau.reshape(_B, 16), bound.reshape(_B, 16))
    nd, db, ds, dc = _k3_call(sc512, cls512, bx)
    det_boxes = jnp.transpose(db[:, :, :_MAX_DET], (0, 2, 1))
    det_scores = ds[:, :_MAX_DET]
    det_classes = dc[:, :_MAX_DET]
    return (nd, det_boxes, det_scores, det_classes)


# speedup vs baseline: 1.0525x; 1.0525x over previous
import functools

import jax
import jax.numpy as jnp
from jax import lax
from jax.experimental import pallas as pl
from jax.experimental.pallas import tpu as pltpu
from jax.experimental.pallas import tpu_sc as plsc

_B = 8
_C = 80
_N = 20000
_MAX_DET = 100
_PRE_TOPK = 512
_IOU_THR = 0.5
_SCORE_THR = 0.25

_NS = 8
_NL = _N // _NS
_LANES = 16


def _float_key(bits):
    return jnp.where(bits >= 0, bits, bits ^ jnp.int32(0x7FFFFFFF))


def _k1a_body(x_ref, maxsc_ref, cls_ref):
    xs = x_ref[0]
    ci8 = lax.broadcasted_iota(jnp.int32, (8, _N), 0)
    m8 = xs[4:12, :]
    c8 = ci8
    for g in range(1, 10):
        blk = xs[4 + 8 * g: 12 + 8 * g, :]
        upd = blk > m8
        c8 = jnp.where(upd, ci8 + 8 * g, c8)
        m8 = jnp.maximum(m8, blk)
    m4 = jnp.maximum(m8[:4], m8[4:])
    c4 = jnp.where(m8[:4] >= m8[4:], c8[:4], c8[4:])
    m2 = jnp.maximum(m4[:2], m4[2:])
    c2 = jnp.where(m4[:2] >= m4[2:], c4[:2], c4[2:])
    m1 = jnp.maximum(m2[:1], m2[1:])
    c1 = jnp.where(m2[:1] >= m2[1:], c2[:1], c2[1:])
    maxsc_ref[0] = m1
    cls_ref[0] = c1


def _k1a_call(x):
    return pl.pallas_call(
        _k1a_body,
        grid=(_B,),
        in_specs=[pl.BlockSpec((1, 4 + _C, _N), lambda b: (b, 0, 0))],
        out_specs=[
            pl.BlockSpec((1, 1, _N), lambda b: (b, 0, 0)),
            pl.BlockSpec((1, 1, _N), lambda b: (b, 0, 0)),
        ],
        out_shape=[
            jax.ShapeDtypeStruct((_B, 1, _N), jnp.float32),
            jax.ShapeDtypeStruct((_B, 1, _N), jnp.int32),
        ],
    )(x)


def _k1b_body(maxsc_ref, tau_ref, bound_ref):
    M = maxsc_ref[...]
    key = _float_key(lax.bitcast_convert_type(M, jnp.int32))
    kmin = jnp.min(key, axis=(1, 2), keepdims=True)
    kmax = jnp.max(key, axis=(1, 2), keepdims=True)

    def cnt_ge(v):
        return jnp.sum(jnp.where(key >= v, 1.0, 0.0), axis=(1, 2),
                       keepdims=True)

    topkf = float(_PRE_TOPK)

    def sbody(_, carry):
        lo, hi = carry
        mid = lo + (hi - lo) // 2
        p = cnt_ge(mid) >= topkf
        return jnp.where(p, mid, lo), jnp.where(p, hi, mid)

    lo, _hi = lax.fori_loop(0, 32, sbody, (kmin, kmax + 1))
    tau = lo
    n_tie = topkf - jnp.sum(jnp.where(key > tau, 1.0, 0.0), axis=(1, 2),
                            keepdims=True)

    flat = (lax.broadcasted_iota(jnp.int32, (_B, _NS, _NL), 1) * _NL
            + lax.broadcasted_iota(jnp.int32, (_B, _NS, _NL), 2))
    eqm = key == tau

    def tbody(_, carry):
        lo2, hi2 = carry
        mid = (lo2 + hi2) // 2
        cnt = jnp.sum(jnp.where(eqm & (flat < mid), 1.0, 0.0), axis=(1, 2),
                      keepdims=True)
        q = cnt >= n_tie
        return jnp.where(q, lo2, mid), jnp.where(q, mid, hi2)

    zero = jnp.zeros((_B, 1, 1), jnp.int32)
    _lo2, bound = lax.fori_loop(0, 15, tbody, (zero, zero + _N))

    tau_bits = _float_key(tau)
    tau_f = lax.bitcast_convert_type(tau_bits, jnp.float32)
    tau_ref[...] = jnp.broadcast_to(tau_f, (_B, 1, 16))
    bound_ref[...] = jnp.broadcast_to(bound, (_B, 1, 16))


def _k1b_call(maxsc):
    return pl.pallas_call(
        _k1b_body,
        out_shape=[
            jax.ShapeDtypeStruct((_B, 1, 16), jnp.float32),
            jax.ShapeDtypeStruct((_B, 1, 16), jnp.int32),
        ],
    )(maxsc)


def _k1_call(x):
    maxsc, cls8 = _k1a_call(x)
    maxsc = maxsc.reshape(_B, _NS, _NL)
    tau, bound = _k1b_call(maxsc)
    return maxsc, cls8, tau, bound


def _k2_body(maxsc_hbm, cls_hbm, x_hbm, tau_hbm, bnd_hbm,
             sc_out, cls_out, bx_out,
             sc_v, cls_v, cx_v, cy_v, w_v, h_v,
             tau_v, bnd_v, idx_v, osc_v, ocls_v, o0, o1, o2, o3):
    c = lax.axis_index("c")
    s = lax.axis_index("s")
    wid = s * 2 + c

    @pl.when(wid < _B)
    def _():
        b = wid
        pltpu.sync_copy(maxsc_hbm.at[b], sc_v)
        pltpu.sync_copy(cls_hbm.at[b], cls_v)
        pltpu.sync_copy(x_hbm.at[b, 0], cx_v)
        pltpu.sync_copy(x_hbm.at[b, 1], cy_v)
        pltpu.sync_copy(x_hbm.at[b, 2], w_v)
        pltpu.sync_copy(x_hbm.at[b, 3], h_v)
        pltpu.sync_copy(tau_hbm.at[b], tau_v)
        pltpu.sync_copy(bnd_hbm.at[b], bnd_v)
        tau = tau_v[...]
        bndf = bnd_v[...].astype(jnp.float32)
        lane = lax.iota(jnp.int32, _LANES)

        def body(i, cur):
            v = sc_v[pl.ds(i * _LANES, _LANES)]
            cl = cls_v[pl.ds(i * _LANES, _LANES)]
            idx = lane + i * _LANES
            idxf = idx.astype(jnp.float32)
            sel = (v > tau) | ((v == tau) & (idxf < bndf))
            csum = plsc.cumsum(sel.astype(jnp.int32))
            pos = csum + (cur - 1)
            plsc.store_scatter(idx_v, [pos], idx, mask=sel)
            plsc.store_scatter(osc_v, [pos], v, mask=sel)
            plsc.store_scatter(ocls_v, [pos], cl, mask=sel)
            return cur + jnp.max(csum)

        lax.fori_loop(0, _N // _LANES, body, jnp.int32(0), unroll=4)

        def gbody(i, _):
            sl = pl.ds(i * _LANES, _LANES)
            ii = idx_v[sl]
            cx = plsc.load_gather(cx_v, [ii])
            cy = plsc.load_gather(cy_v, [ii])
            w = plsc.load_gather(w_v, [ii])
            h = plsc.load_gather(h_v, [ii])
            o0[sl] = cx - w * 0.5
            o1[sl] = cy - h * 0.5
            o2[sl] = cx + w * 0.5
            o3[sl] = cy + h * 0.5
            return 0

        lax.fori_loop(0, _PRE_TOPK // _LANES, gbody, 0, unroll=4)

        pltpu.sync_copy(osc_v, sc_out.at[b])
        pltpu.sync_copy(ocls_v, cls_out.at[b])
        pltpu.sync_copy(o0, bx_out.at[b, 0])
        pltpu.sync_copy(o1, bx_out.at[b, 1])
        pltpu.sync_copy(o2, bx_out.at[b, 2])
        pltpu.sync_copy(o3, bx_out.at[b, 3])


def _k2_call(maxsc, cls8, x, tau, bound):
    mesh = plsc.VectorSubcoreMesh(core_axis_name="c", subcore_axis_name="s")
    f = functools.partial(
        pl.kernel,
        out_type=[
            jax.ShapeDtypeStruct((_B, _PRE_TOPK), jnp.float32),
            jax.ShapeDtypeStruct((_B, _PRE_TOPK), jnp.int32),
            jax.ShapeDtypeStruct((_B, 4, _PRE_TOPK), jnp.float32),
        ],
        mesh=mesh,
        compiler_params=pltpu.CompilerParams(needs_layout_passes=False),
        scratch_types=[
            pltpu.VMEM((_N,), jnp.float32),
            pltpu.VMEM((_N,), jnp.int32),
            pltpu.VMEM((_N,), jnp.float32),
            pltpu.VMEM((_N,), jnp.float32),
            pltpu.VMEM((_N,), jnp.float32),
            pltpu.VMEM((_N,), jnp.float32),
            pltpu.VMEM((16,), jnp.float32),
            pltpu.VMEM((16,), jnp.int32),
            pltpu.VMEM((_PRE_TOPK,), jnp.int32),
            pltpu.VMEM((_PRE_TOPK,), jnp.float32),
            pltpu.VMEM((_PRE_TOPK,), jnp.int32),
            pltpu.VMEM((_PRE_TOPK,), jnp.float32),
            pltpu.VMEM((_PRE_TOPK,), jnp.float32),
            pltpu.VMEM((_PRE_TOPK,), jnp.float32),
            pltpu.VMEM((_PRE_TOPK,), jnp.float32),
        ],
    )(_k2_body)
    return f(maxsc, cls8, x, tau, bound)


def _k3_body(sc_ref, cls_ref, bx_ref, nd_ref, db_ref, ds_ref, dc_ref):
    sc = sc_ref[...]
    clsf = cls_ref[...].astype(jnp.float32)
    x1 = bx_ref[:, 0, :]
    y1 = bx_ref[:, 1, :]
    x2 = bx_ref[:, 2, :]
    y2 = bx_ref[:, 3, :]
    area = jnp.clip(x2 - x1, 0.0) * jnp.clip(y2 - y1, 0.0)
    sc_w0 = jnp.where(sc > _SCORE_THR, sc, -1.0)
    iota = lax.broadcasted_iota(jnp.int32, (_B, _PRE_TOPK), 1)
    iota_o = lax.broadcasted_iota(jnp.int32, (_B, 128), 1)
    zf = jnp.zeros((_B, 128), jnp.float32)

    def body(i, carry):
        sc_w, cnt, a1o, a2o, a3o, a4o, aso, aco = carry
        m = jnp.max(sc_w, axis=1, keepdims=True)
        eq = sc_w == m
        j = jnp.min(jnp.where(eq, iota, _PRE_TOPK), axis=1, keepdims=True)
        oh = iota == j
        bx1 = jnp.sum(jnp.where(oh, x1, 0.0), axis=1, keepdims=True)
        by1 = jnp.sum(jnp.where(oh, y1, 0.0), axis=1, keepdims=True)
        bx2 = jnp.sum(jnp.where(oh, x2, 0.0), axis=1, keepdims=True)
        by2 = jnp.sum(jnp.where(oh, y2, 0.0), axis=1, keepdims=True)
        bcf = jnp.sum(jnp.where(oh, clsf, 0.0), axis=1, keepdims=True)
        keep = m > _SCORE_THR
        ohw = (iota_o == i) & keep
        a1o = jnp.where(ohw, bx1, a1o)
        a2o = jnp.where(ohw, by1, a2o)
        a3o = jnp.where(ohw, bx2, a3o)
        a4o = jnp.where(ohw, by2, a4o)
        aso = jnp.where(ohw, m, aso)
        aco = jnp.where(ohw, bcf, aco)
        cnt = cnt + keep.astype(jnp.int32)
        ix1 = jnp.maximum(bx1, x1)
        iy1 = jnp.maximum(by1, y1)
        ix2 = jnp.minimum(bx2, x2)
        iy2 = jnp.minimum(by2, y2)
        inter = jnp.clip(ix2 - ix1, 0.0) * jnp.clip(iy2 - iy1, 0.0)
        a1 = jnp.clip(bx2 - bx1, 0.0) * jnp.clip(by2 - by1, 0.0)
        iou = inter / (a1 + area - inter + 1e-9)
        supp = (iou > _IOU_THR) & (clsf == bcf)
        sc_w = jnp.where(supp | oh, -1.0, sc_w)
        return sc_w, cnt, a1o, a2o, a3o, a4o, aso, aco

    init = (sc_w0, jnp.zeros((_B, 1), jnp.int32), zf, zf, zf, zf, zf,
            zf - 1.0)
    _, cnt, a1o, a2o, a3o, a4o, aso, aco = lax.fori_loop(
        0, _MAX_DET, body, init)
    nd_ref[...] = cnt
    db_ref[...] = jnp.concatenate(
        [a1o[:, None, :], a2o[:, None, :], a3o[:, None, :], a4o[:, None, :]],
        axis=1)
    ds_ref[...] = aso
    dc_ref[...] = aco.astype(jnp.int32)


def _k3_call(sc512, cls512, bx):
    return pl.pallas_call(
        _k3_body,
        out_shape=[
            jax.ShapeDtypeStruct((_B, 1), jnp.int32),
            jax.ShapeDtypeStruct((_B, 4, 128), jnp.float32),
            jax.ShapeDtypeStruct((_B, 128), jnp.float32),
            jax.ShapeDtypeStruct((_B, 128), jnp.int32),
        ],
    )(sc512, cls512, bx)


def kernel(x):
    maxsc, cls8, tau, bound = _k1_call(x)
    sc512, cls512, bx = _k2_call(
        maxsc.reshape(_B, _N), cls8.reshape(_B, _N), x,
        tau.reshape(_B, 16), bound.reshape(_B, 16))
    nd, db, ds, dc = _k3_call(sc512, cls512, bx)
    det_boxes = jnp.transpose(db[:, :, :_MAX_DET], (0, 2, 1))
    det_scores = ds[:, :_MAX_DET]
    det_classes = dc[:, :_MAX_DET]
    return (nd, det_boxes, det_scores, det_classes)


# speedup vs baseline: 1.0554x; 1.0028x over previous
import functools

import jax
import jax.numpy as jnp
from jax import lax
from jax.experimental import pallas as pl
from jax.experimental.pallas import tpu as pltpu
from jax.experimental.pallas import tpu_sc as plsc

_B = 8
_C = 80
_N = 20000
_MAX_DET = 100
_PRE_TOPK = 512
_IOU_THR = 0.5
_SCORE_THR = 0.25

_NS = 8
_NL = _N // _NS
_LANES = 16


def _float_key(bits):
    return jnp.where(bits >= 0, bits, bits ^ jnp.int32(0x7FFFFFFF))


def _k1a_body(x_ref, maxsc_ref, cls_ref):
    xs = x_ref[0]
    ci8 = lax.broadcasted_iota(jnp.int32, (8, _N), 0)
    m8 = xs[4:12, :]
    c8 = ci8
    for g in range(1, 10):
        blk = xs[4 + 8 * g: 12 + 8 * g, :]
        upd = blk > m8
        c8 = jnp.where(upd, ci8 + 8 * g, c8)
        m8 = jnp.maximum(m8, blk)
    m4 = jnp.maximum(m8[:4], m8[4:])
    c4 = jnp.where(m8[:4] >= m8[4:], c8[:4], c8[4:])
    m2 = jnp.maximum(m4[:2], m4[2:])
    c2 = jnp.where(m4[:2] >= m4[2:], c4[:2], c4[2:])
    m1 = jnp.maximum(m2[:1], m2[1:])
    c1 = jnp.where(m2[:1] >= m2[1:], c2[:1], c2[1:])
    maxsc_ref[0] = m1
    cls_ref[0] = c1


def _k1a_call(x):
    return pl.pallas_call(
        _k1a_body,
        grid=(_B,),
        in_specs=[pl.BlockSpec((1, 4 + _C, _N), lambda b: (b, 0, 0))],
        out_specs=[
            pl.BlockSpec((1, 1, _N), lambda b: (b, 0, 0)),
            pl.BlockSpec((1, 1, _N), lambda b: (b, 0, 0)),
        ],
        out_shape=[
            jax.ShapeDtypeStruct((_B, 1, _N), jnp.float32),
            jax.ShapeDtypeStruct((_B, 1, _N), jnp.int32),
        ],
    )(x)


def _k1b_body(maxsc_ref, tau_ref, bound_ref):
    M = maxsc_ref[...]
    key = _float_key(lax.bitcast_convert_type(M, jnp.int32))
    kmin = jnp.min(key, axis=(1, 2), keepdims=True)
    kmax = jnp.max(key, axis=(1, 2), keepdims=True)

    def cnt_ge(v):
        return jnp.sum(jnp.where(key >= v, 1.0, 0.0), axis=(1, 2),
                       keepdims=True)

    topkf = float(_PRE_TOPK)

    def sbody(_, carry):
        lo, hi = carry
        mid = lo + (hi - lo) // 2
        p = cnt_ge(mid) >= topkf
        return jnp.where(p, mid, lo), jnp.where(p, hi, mid)

    lo, _hi = lax.fori_loop(0, 32, sbody, (kmin, kmax + 1))
    tau = lo
    n_tie = topkf - jnp.sum(jnp.where(key > tau, 1.0, 0.0), axis=(1, 2),
                            keepdims=True)

    flat = (lax.broadcasted_iota(jnp.int32, (_B, _NS, _NL), 1) * _NL
            + lax.broadcasted_iota(jnp.int32, (_B, _NS, _NL), 2))
    eqm = key == tau

    def tbody(_, carry):
        lo2, hi2 = carry
        mid = (lo2 + hi2) // 2
        cnt = jnp.sum(jnp.where(eqm & (flat < mid), 1.0, 0.0), axis=(1, 2),
                      keepdims=True)
        q = cnt >= n_tie
        return jnp.where(q, lo2, mid), jnp.where(q, mid, hi2)

    zero = jnp.zeros((_B, 1, 1), jnp.int32)
    _lo2, bound = lax.fori_loop(0, 15, tbody, (zero, zero + _N))

    tau_bits = _float_key(tau)
    tau_f = lax.bitcast_convert_type(tau_bits, jnp.float32)
    tau_ref[...] = jnp.broadcast_to(tau_f, (_B, 1, 16))
    bound_ref[...] = jnp.broadcast_to(bound, (_B, 1, 16))


def _k1b_call(maxsc):
    return pl.pallas_call(
        _k1b_body,
        out_shape=[
            jax.ShapeDtypeStruct((_B, 1, 16), jnp.float32),
            jax.ShapeDtypeStruct((_B, 1, 16), jnp.int32),
        ],
    )(maxsc)


def _k1_call(x):
    maxsc, cls8 = _k1a_call(x)
    maxsc = maxsc.reshape(_B, _NS, _NL)
    tau, bound = _k1b_call(maxsc)
    return maxsc, cls8, tau, bound


def _k2_body(maxsc_hbm, cls_hbm, x_hbm, tau_hbm, bnd_hbm,
             sc_out, cls_out, bx_out,
             sc_v, cls_v, cx_v, cy_v, w_v, h_v,
             tau_v, bnd_v, idx_v, osc_v, ocls_v, o0, o1, o2, o3):
    c = lax.axis_index("c")
    s = lax.axis_index("s")
    wid = s * 2 + c

    @pl.when(wid < _B)
    def _():
        b = wid
        pltpu.sync_copy(maxsc_hbm.at[b], sc_v)
        pltpu.sync_copy(cls_hbm.at[b], cls_v)
        pltpu.sync_copy(x_hbm.at[b, 0], cx_v)
        pltpu.sync_copy(x_hbm.at[b, 1], cy_v)
        pltpu.sync_copy(x_hbm.at[b, 2], w_v)
        pltpu.sync_copy(x_hbm.at[b, 3], h_v)
        pltpu.sync_copy(tau_hbm.at[b], tau_v)
        pltpu.sync_copy(bnd_hbm.at[b], bnd_v)
        tau = tau_v[...]
        bndf = bnd_v[...].astype(jnp.float32)
        lane = lax.iota(jnp.int32, _LANES)

        def body(i, cur):
            v = sc_v[pl.ds(i * _LANES, _LANES)]
            cl = cls_v[pl.ds(i * _LANES, _LANES)]
            idx = lane + i * _LANES
            idxf = idx.astype(jnp.float32)
            sel = (v > tau) | ((v == tau) & (idxf < bndf))
            csum = plsc.cumsum(sel.astype(jnp.int32))
            pos = csum + (cur - 1)
            plsc.store_scatter(idx_v, [pos], idx, mask=sel)
            plsc.store_scatter(osc_v, [pos], v, mask=sel)
            plsc.store_scatter(ocls_v, [pos], cl, mask=sel)
            return cur + plsc.all_reduce_population_count(sel)

        lax.fori_loop(0, _N // _LANES, body,
                      jnp.zeros((_LANES,), jnp.int32), unroll=4)

        def gbody(i, _):
            sl = pl.ds(i * _LANES, _LANES)
            ii = idx_v[sl]
            cx = plsc.load_gather(cx_v, [ii])
            cy = plsc.load_gather(cy_v, [ii])
            w = plsc.load_gather(w_v, [ii])
            h = plsc.load_gather(h_v, [ii])
            o0[sl] = cx - w * 0.5
            o1[sl] = cy - h * 0.5
            o2[sl] = cx + w * 0.5
            o3[sl] = cy + h * 0.5
            return 0

        lax.fori_loop(0, _PRE_TOPK // _LANES, gbody, 0, unroll=4)

        pltpu.sync_copy(osc_v, sc_out.at[b])
        pltpu.sync_copy(ocls_v, cls_out.at[b])
        pltpu.sync_copy(o0, bx_out.at[b, 0])
        pltpu.sync_copy(o1, bx_out.at[b, 1])
        pltpu.sync_copy(o2, bx_out.at[b, 2])
        pltpu.sync_copy(o3, bx_out.at[b, 3])


def _k2_call(maxsc, cls8, x, tau, bound):
    mesh = plsc.VectorSubcoreMesh(core_axis_name="c", subcore_axis_name="s")
    f = functools.partial(
        pl.kernel,
        out_type=[
            jax.ShapeDtypeStruct((_B, _PRE_TOPK), jnp.float32),
            jax.ShapeDtypeStruct((_B, _PRE_TOPK), jnp.int32),
            jax.ShapeDtypeStruct((_B, 4, _PRE_TOPK), jnp.float32),
        ],
        mesh=mesh,
        compiler_params=pltpu.CompilerParams(needs_layout_passes=False),
        scratch_types=[
            pltpu.VMEM((_N,), jnp.float32),
            pltpu.VMEM((_N,), jnp.int32),
            pltpu.VMEM((_N,), jnp.float32),
            pltpu.VMEM((_N,), jnp.float32),
            pltpu.VMEM((_N,), jnp.float32),
            pltpu.VMEM((_N,), jnp.float32),
            pltpu.VMEM((16,), jnp.float32),
            pltpu.VMEM((16,), jnp.int32),
            pltpu.VMEM((_PRE_TOPK,), jnp.int32),
            pltpu.VMEM((_PRE_TOPK,), jnp.float32),
            pltpu.VMEM((_PRE_TOPK,), jnp.int32),
            pltpu.VMEM((_PRE_TOPK,), jnp.float32),
            pltpu.VMEM((_PRE_TOPK,), jnp.float32),
            pltpu.VMEM((_PRE_TOPK,), jnp.float32),
            pltpu.VMEM((_PRE_TOPK,), jnp.float32),
        ],
    )(_k2_body)
    return f(maxsc, cls8, x, tau, bound)


def _k3_body(sc_ref, cls_ref, bx_ref, nd_ref, db_ref, ds_ref, dc_ref):
    sc = sc_ref[...]
    clsf = cls_ref[...].astype(jnp.float32)
    x1 = bx_ref[:, 0, :]
    y1 = bx_ref[:, 1, :]
    x2 = bx_ref[:, 2, :]
    y2 = bx_ref[:, 3, :]
    area = jnp.clip(x2 - x1, 0.0) * jnp.clip(y2 - y1, 0.0)
    sc_w0 = jnp.where(sc > _SCORE_THR, sc, -1.0)
    iota = lax.broadcasted_iota(jnp.int32, (_B, _PRE_TOPK), 1)
    iota_o = lax.broadcasted_iota(jnp.int32, (_B, 128), 1)
    zf = jnp.zeros((_B, 128), jnp.float32)

    def body(i, carry):
        sc_w, cnt, a1o, a2o, a3o, a4o, aso, aco = carry
        m = jnp.max(sc_w, axis=1, keepdims=True)
        eq = sc_w == m
        j = jnp.min(jnp.where(eq, iota, _PRE_TOPK), axis=1, keepdims=True)
        oh = iota == j
        bx1 = jnp.sum(jnp.where(oh, x1, 0.0), axis=1, keepdims=True)
        by1 = jnp.sum(jnp.where(oh, y1, 0.0), axis=1, keepdims=True)
        bx2 = jnp.sum(jnp.where(oh, x2, 0.0), axis=1, keepdims=True)
        by2 = jnp.sum(jnp.where(oh, y2, 0.0), axis=1, keepdims=True)
        bcf = jnp.sum(jnp.where(oh, clsf, 0.0), axis=1, keepdims=True)
        keep = m > _SCORE_THR
        ohw = (iota_o == i) & keep
        a1o = jnp.where(ohw, bx1, a1o)
        a2o = jnp.where(ohw, by1, a2o)
        a3o = jnp.where(ohw, bx2, a3o)
        a4o = jnp.where(ohw, by2, a4o)
        aso = jnp.where(ohw, m, aso)
        aco = jnp.where(ohw, bcf, aco)
        cnt = cnt + keep.astype(jnp.int32)
        ix1 = jnp.maximum(bx1, x1)
        iy1 = jnp.maximum(by1, y1)
        ix2 = jnp.minimum(bx2, x2)
        iy2 = jnp.minimum(by2, y2)
        inter = jnp.clip(ix2 - ix1, 0.0) * jnp.clip(iy2 - iy1, 0.0)
        a1 = jnp.clip(bx2 - bx1, 0.0) * jnp.clip(by2 - by1, 0.0)
        iou = inter / (a1 + area - inter + 1e-9)
        supp = (iou > _IOU_THR) & (clsf == bcf)
        sc_w = jnp.where(supp | oh, -1.0, sc_w)
        return sc_w, cnt, a1o, a2o, a3o, a4o, aso, aco

    init = (sc_w0, jnp.zeros((_B, 1), jnp.int32), zf, zf, zf, zf, zf,
            zf - 1.0)
    _, cnt, a1o, a2o, a3o, a4o, aso, aco = lax.fori_loop(
        0, _MAX_DET, body, init)
    nd_ref[...] = cnt
    db_ref[...] = jnp.concatenate(
        [a1o[:, None, :], a2o[:, None, :], a3o[:, None, :], a4o[:, None, :]],
        axis=1)
    ds_ref[...] = aso
    dc_ref[...] = aco.astype(jnp.int32)


def _k3_call(sc512, cls512, bx):
    return pl.pallas_call(
        _k3_body,
        out_shape=[
            jax.ShapeDtypeStruct((_B, 1), jnp.int32),
            jax.ShapeDtypeStruct((_B, 4, 128), jnp.float32),
            jax.ShapeDtypeStruct((_B, 128), jnp.float32),
            jax.ShapeDtypeStruct((_B, 128), jnp.int32),
        ],
    )(sc512, cls512, bx)


def kernel(x):
    maxsc, cls8, tau, bound = _k1_call(x)
    sc512, cls512, bx = _k2_call(
        maxsc.reshape(_B, _N), cls8.reshape(_B, _N), x,
        tau.reshape(_B, 16), bound.reshape(_B, 16))
    nd, db, ds, dc = _k3_call(sc512, cls512, bx)
    det_boxes = jnp.transpose(db[:, :, :_MAX_DET], (0, 2, 1))
    det_scores = ds[:, :_MAX_DET]
    det_classes = dc[:, :_MAX_DET]
    return (nd, det_boxes, det_scores, det_classes)


# speedup vs baseline: 1.0598x; 1.0041x over previous
import functools

import jax
import jax.numpy as jnp
from jax import lax
from jax.experimental import pallas as pl
from jax.experimental.pallas import tpu as pltpu
from jax.experimental.pallas import tpu_sc as plsc

_B = 8
_C = 80
_N = 20000
_MAX_DET = 100
_PRE_TOPK = 512
_IOU_THR = 0.5
_SCORE_THR = 0.25

_NS = 8
_NL = _N // _NS
_LANES = 16


def _float_key(bits):
    return jnp.where(bits >= 0, bits, bits ^ jnp.int32(0x7FFFFFFF))


def _k1a_body(x_ref, maxsc_ref, cls_ref):
    xs = x_ref[0]
    ci8 = lax.broadcasted_iota(jnp.int32, (8, _N), 0)
    m8 = xs[4:12, :]
    c8 = ci8
    for g in range(1, 10):
        blk = xs[4 + 8 * g: 12 + 8 * g, :]
        upd = blk > m8
        c8 = jnp.where(upd, ci8 + 8 * g, c8)
        m8 = jnp.maximum(m8, blk)
    m4 = jnp.maximum(m8[:4], m8[4:])
    c4 = jnp.where(m8[:4] >= m8[4:], c8[:4], c8[4:])
    m2 = jnp.maximum(m4[:2], m4[2:])
    c2 = jnp.where(m4[:2] >= m4[2:], c4[:2], c4[2:])
    m1 = jnp.maximum(m2[:1], m2[1:])
    c1 = jnp.where(m2[:1] >= m2[1:], c2[:1], c2[1:])
    maxsc_ref[0] = m1
    cls_ref[0] = c1


def _k1a_call(x):
    return pl.pallas_call(
        _k1a_body,
        grid=(_B,),
        in_specs=[pl.BlockSpec((1, 4 + _C, _N), lambda b: (b, 0, 0))],
        out_specs=[
            pl.BlockSpec((1, 1, _N), lambda b: (b, 0, 0)),
            pl.BlockSpec((1, 1, _N), lambda b: (b, 0, 0)),
        ],
        out_shape=[
            jax.ShapeDtypeStruct((_B, 1, _N), jnp.float32),
            jax.ShapeDtypeStruct((_B, 1, _N), jnp.int32),
        ],
    )(x)


def _k1b_body(maxsc_ref, tau_ref, bound_ref):
    M = maxsc_ref[...]
    key = _float_key(lax.bitcast_convert_type(M, jnp.int32))
    kmin = jnp.min(key, axis=(1, 2), keepdims=True)
    kmax = jnp.max(key, axis=(1, 2), keepdims=True)

    def cnt_ge(v):
        return jnp.sum(jnp.where(key >= v, 1.0, 0.0), axis=(1, 2),
                       keepdims=True)

    topkf = float(_PRE_TOPK)

    def sbody(_, carry):
        lo, hi = carry
        w = hi - lo
        q1 = lo + w // 4
        q2 = lo + w // 2
        q3 = q2 + w // 4
        c1 = cnt_ge(q1) >= topkf
        c2 = cnt_ge(q2) >= topkf
        c3 = cnt_ge(q3) >= topkf
        lo2 = jnp.where(c3, q3, jnp.where(c2, q2, jnp.where(c1, q1, lo)))
        hi2 = jnp.where(c3, hi, jnp.where(c2, q3, jnp.where(c1, q2, q1)))
        return lo2, hi2

    lo, _hi = lax.fori_loop(0, 16, sbody, (kmin, kmax + 1))
    tau = lo
    n_tie = topkf - jnp.sum(jnp.where(key > tau, 1.0, 0.0), axis=(1, 2),
                            keepdims=True)

    flat = (lax.broadcasted_iota(jnp.int32, (_B, _NS, _NL), 1) * _NL
            + lax.broadcasted_iota(jnp.int32, (_B, _NS, _NL), 2))
    eqm = key == tau

    def cnt_lt(v):
        return jnp.sum(jnp.where(eqm & (flat < v), 1.0, 0.0), axis=(1, 2),
                       keepdims=True)

    def tbody(_, carry):
        lo2, hi2 = carry
        w = hi2 - lo2
        q1 = lo2 + w // 4
        q2 = lo2 + w // 2
        q3 = q2 + w // 4
        c1 = cnt_lt(q1) >= n_tie
        c2 = cnt_lt(q2) >= n_tie
        c3 = cnt_lt(q3) >= n_tie
        lo3 = jnp.where(c1, lo2, jnp.where(c2, q1, jnp.where(c3, q2, q3)))
        hi3 = jnp.where(c1, q1, jnp.where(c2, q2, jnp.where(c3, q3, hi2)))
        return lo3, hi3

    zero = jnp.zeros((_B, 1, 1), jnp.int32)
    _lo2, bound = lax.fori_loop(0, 8, tbody, (zero, zero + _N))

    tau_bits = _float_key(tau)
    tau_f = lax.bitcast_convert_type(tau_bits, jnp.float32)
    tau_ref[...] = jnp.broadcast_to(tau_f, (_B, 1, 16))
    bound_ref[...] = jnp.broadcast_to(bound, (_B, 1, 16))


def _k1b_call(maxsc):
    return pl.pallas_call(
        _k1b_body,
        out_shape=[
            jax.ShapeDtypeStruct((_B, 1, 16), jnp.float32),
            jax.ShapeDtypeStruct((_B, 1, 16), jnp.int32),
        ],
    )(maxsc)


def _k1_call(x):
    maxsc, cls8 = _k1a_call(x)
    maxsc = maxsc.reshape(_B, _NS, _NL)
    tau, bound = _k1b_call(maxsc)
    return maxsc, cls8, tau, bound


def _k2_body(maxsc_hbm, cls_hbm, x_hbm, tau_hbm, bnd_hbm,
             sc_out, cls_out, bx_out,
             sc_v, cls_v, cx_v, cy_v, w_v, h_v,
             tau_v, bnd_v, idx_v, osc_v, ocls_v, o0, o1, o2, o3):
    c = lax.axis_index("c")
    s = lax.axis_index("s")
    wid = s * 2 + c

    @pl.when(wid < _B)
    def _():
        b = wid
        pltpu.sync_copy(maxsc_hbm.at[b], sc_v)
        pltpu.sync_copy(cls_hbm.at[b], cls_v)
        pltpu.sync_copy(x_hbm.at[b, 0], cx_v)
        pltpu.sync_copy(x_hbm.at[b, 1], cy_v)
        pltpu.sync_copy(x_hbm.at[b, 2], w_v)
        pltpu.sync_copy(x_hbm.at[b, 3], h_v)
        pltpu.sync_copy(tau_hbm.at[b], tau_v)
        pltpu.sync_copy(bnd_hbm.at[b], bnd_v)
        tau = tau_v[...]
        bndf = bnd_v[...].astype(jnp.float32)
        lane = lax.iota(jnp.int32, _LANES)

        def body(i, cur):
            v = sc_v[pl.ds(i * _LANES, _LANES)]
            cl = cls_v[pl.ds(i * _LANES, _LANES)]
            idx = lane + i * _LANES
            idxf = idx.astype(jnp.float32)
            sel = (v > tau) | ((v == tau) & (idxf < bndf))
            csum = plsc.cumsum(sel.astype(jnp.int32))
            pos = csum + (cur - 1)
            plsc.store_scatter(idx_v, [pos], idx, mask=sel)
            plsc.store_scatter(osc_v, [pos], v, mask=sel)
            plsc.store_scatter(ocls_v, [pos], cl, mask=sel)
            return cur + plsc.all_reduce_population_count(sel)

        lax.fori_loop(0, _N // _LANES, body,
                      jnp.zeros((_LANES,), jnp.int32), unroll=4)

        def gbody(i, _):
            sl = pl.ds(i * _LANES, _LANES)
            ii = idx_v[sl]
            cx = plsc.load_gather(cx_v, [ii])
            cy = plsc.load_gather(cy_v, [ii])
            w = plsc.load_gather(w_v, [ii])
            h = plsc.load_gather(h_v, [ii])
            o0[sl] = cx - w * 0.5
            o1[sl] = cy - h * 0.5
            o2[sl] = cx + w * 0.5
            o3[sl] = cy + h * 0.5
            return 0

        lax.fori_loop(0, _PRE_TOPK // _LANES, gbody, 0, unroll=4)

        pltpu.sync_copy(osc_v, sc_out.at[b])
        pltpu.sync_copy(ocls_v, cls_out.at[b])
        pltpu.sync_copy(o0, bx_out.at[b, 0])
        pltpu.sync_copy(o1, bx_out.at[b, 1])
        pltpu.sync_copy(o2, bx_out.at[b, 2])
        pltpu.sync_copy(o3, bx_out.at[b, 3])


def _k2_call(maxsc, cls8, x, tau, bound):
    mesh = plsc.VectorSubcoreMesh(core_axis_name="c", subcore_axis_name="s")
    f = functools.partial(
        pl.kernel,
        out_type=[
            jax.ShapeDtypeStruct((_B, _PRE_TOPK), jnp.float32),
            jax.ShapeDtypeStruct((_B, _PRE_TOPK), jnp.int32),
            jax.ShapeDtypeStruct((_B, 4, _PRE_TOPK), jnp.float32),
        ],
        mesh=mesh,
        compiler_params=pltpu.CompilerParams(needs_layout_passes=False),
        scratch_types=[
            pltpu.VMEM((_N,), jnp.float32),
            pltpu.VMEM((_N,), jnp.int32),
            pltpu.VMEM((_N,), jnp.float32),
            pltpu.VMEM((_N,), jnp.float32),
            pltpu.VMEM((_N,), jnp.float32),
            pltpu.VMEM((_N,), jnp.float32),
            pltpu.VMEM((16,), jnp.float32),
            pltpu.VMEM((16,), jnp.int32),
            pltpu.VMEM((_PRE_TOPK,), jnp.int32),
            pltpu.VMEM((_PRE_TOPK,), jnp.float32),
            pltpu.VMEM((_PRE_TOPK,), jnp.int32),
            pltpu.VMEM((_PRE_TOPK,), jnp.float32),
            pltpu.VMEM((_PRE_TOPK,), jnp.float32),
            pltpu.VMEM((_PRE_TOPK,), jnp.float32),
            pltpu.VMEM((_PRE_TOPK,), jnp.float32),
        ],
    )(_k2_body)
    return f(maxsc, cls8, x, tau, bound)


def _k3_body(sc_ref, cls_ref, bx_ref, nd_ref, db_ref, ds_ref, dc_ref):
    sc = sc_ref[...]
    clsf = cls_ref[...].astype(jnp.float32)
    x1 = bx_ref[:, 0, :]
    y1 = bx_ref[:, 1, :]
    x2 = bx_ref[:, 2, :]
    y2 = bx_ref[:, 3, :]
    area = jnp.clip(x2 - x1, 0.0) * jnp.clip(y2 - y1, 0.0)
    sc_w0 = jnp.where(sc > _SCORE_THR, sc, -1.0)
    iota = lax.broadcasted_iota(jnp.int32, (_B, _PRE_TOPK), 1)
    iota_o = lax.broadcasted_iota(jnp.int32, (_B, 128), 1)
    zf = jnp.zeros((_B, 128), jnp.float32)

    def body(i, carry):
        sc_w, cnt, a1o, a2o, a3o, a4o, aso, aco = carry
        m = jnp.max(sc_w, axis=1, keepdims=True)
        eq = sc_w == m
        j = jnp.min(jnp.where(eq, iota, _PRE_TOPK), axis=1, keepdims=True)
        oh = iota == j
        bx1 = jnp.sum(jnp.where(oh, x1, 0.0), axis=1, keepdims=True)
        by1 = jnp.sum(jnp.where(oh, y1, 0.0), axis=1, keepdims=True)
        bx2 = jnp.sum(jnp.where(oh, x2, 0.0), axis=1, keepdims=True)
        by2 = jnp.sum(jnp.where(oh, y2, 0.0), axis=1, keepdims=True)
        bcf = jnp.sum(jnp.where(oh, clsf, 0.0), axis=1, keepdims=True)
        keep = m > _SCORE_THR
        ohw = (iota_o == i) & keep
        a1o = jnp.where(ohw, bx1, a1o)
        a2o = jnp.where(ohw, by1, a2o)
        a3o = jnp.where(ohw, bx2, a3o)
        a4o = jnp.where(ohw, by2, a4o)
        aso = jnp.where(ohw, m, aso)
        aco = jnp.where(ohw, bcf, aco)
        cnt = cnt + keep.astype(jnp.int32)
        ix1 = jnp.maximum(bx1, x1)
        iy1 = jnp.maximum(by1, y1)
        ix2 = jnp.minimum(bx2, x2)
        iy2 = jnp.minimum(by2, y2)
        inter = jnp.clip(ix2 - ix1, 0.0) * jnp.clip(iy2 - iy1, 0.0)
        a1 = jnp.clip(bx2 - bx1, 0.0) * jnp.clip(by2 - by1, 0.0)
        iou = inter / (a1 + area - inter + 1e-9)
        supp = (iou > _IOU_THR) & (clsf == bcf)
        sc_w = jnp.where(supp | oh, -1.0, sc_w)
        return sc_w, cnt, a1o, a2o, a3o, a4o, aso, aco

    init = (sc_w0, jnp.zeros((_B, 1), jnp.int32), zf, zf, zf, zf, zf,
            zf - 1.0)
    _, cnt, a1o, a2o, a3o, a4o, aso, aco = lax.fori_loop(
        0, _MAX_DET, body, init)
    nd_ref[...] = cnt
    db_ref[...] = jnp.concatenate(
        [a1o[:, None, :], a2o[:, None, :], a3o[:, None, :], a4o[:, None, :]],
        axis=1)
    ds_ref[...] = aso
    dc_ref[...] = aco.astype(jnp.int32)


def _k3_call(sc512, cls512, bx):
    return pl.pallas_call(
        _k3_body,
        out_shape=[
            jax.ShapeDtypeStruct((_B, 1), jnp.int32),
            jax.ShapeDtypeStruct((_B, 4, 128), jnp.float32),
            jax.ShapeDtypeStruct((_B, 128), jnp.float32),
            jax.ShapeDtypeStruct((_B, 128), jnp.int32),
        ],
    )(sc512, cls512, bx)


def kernel(x):
    maxsc, cls8, tau, bound = _k1_call(x)
    sc512, cls512, bx = _k2_call(
        maxsc.reshape(_B, _N), cls8.reshape(_B, _N), x,
        tau.reshape(_B, 16), bound.reshape(_B, 16))
    nd, db, ds, dc = _k3_call(sc512, cls512, bx)
    det_boxes = jnp.transpose(db[:, :, :_MAX_DET], (0, 2, 1))
    det_scores = ds[:, :_MAX_DET]
    det_classes = dc[:, :_MAX_DET]
    return (nd, det_boxes, det_scores, det_classes)
